# baseline (speedup 1.0000x reference)
import jax
import jax.numpy as jnp
from jax import lax
from jax.experimental import pallas as pl

C = 192
H = 8
HD = C // H
IMG = 224
N = IMG * IMG
R = 8
T = R * IMG
NT = IMG // R
PAD = 8
W_EXT = T + 2 * IMG + 2 * PAD
CP = 256
NEG = -1e30

F32 = jnp.float32
BF16 = jnp.bfloat16


def _bdot(a, b):
    return jnp.dot(a, b, preferred_element_type=F32)


def _pass1_kernel(xc_ref, xt_ref, xb_ref, wlt_ref, bl_ref, wqt_ref, w9_ref,
                  wg1t_ref, bg1_ref, wg2_ref, bg2_ref,
                  q_ref, k_ref, v_ref, sqq_ref, sqk_ref, gs_ref):
    i = pl.program_id(0)
    zpad = jnp.zeros((PAD, C), BF16)
    xe = jnp.concatenate([zpad, xt_ref[...].astype(BF16),
                          xc_ref[...].astype(BF16),
                          xb_ref[...].astype(BF16), zpad], axis=0)
    x1 = _bdot(xe, wlt_ref[...]) + bl_ref[...]
    pre = _bdot(x1.astype(BF16), wqt_ref[...])
    rows = lax.broadcasted_iota(jnp.int32, (W_EXT, 1), 0)
    top_ok = (rows >= PAD + IMG) | (i > 0)
    bot_ok = (rows < W_EXT - PAD - IMG) | (i < NT - 1)
    pad_ok = (rows >= PAD) & (rows < W_EXT - PAD)
    pre = pre * (top_ok & bot_ok & pad_ok).astype(F32)

    preb = pre.astype(BF16).astype(F32)
    shift = {dx: preb[PAD + dx:PAD + dx + T + 2 * IMG, :] for dx in (-1, 0, 1)}
    col = lax.broadcasted_iota(jnp.int32, (T, 1), 0) % IMG
    lm = (col != 0).astype(F32)
    rm = (col != IMG - 1).astype(F32)
    w9 = w9_ref[...]

    def tap(dy, dx):
        j = (dy + 1) * 3 + (dx + 1)
        s = IMG + dy * IMG
        t = w9[j:j + 1, :] * shift[dx][s:s + T, :]
        if dx == -1:
            t = t * lm
        elif dx == 1:
            t = t * rm
        return t

    y = tap(-1, -1)
    for dy, dx in [(-1, 0), (-1, 1), (0, -1), (0, 0), (0, 1),
                   (1, -1), (1, 0), (1, 1)]:
        y = y + tap(dy, dx)

    q = y[:, 0:C]
    k = y[:, CP:CP + C]
    q_ref[...] = q
    k_ref[...] = k
    v_ref[...] = y[:, 2 * CP:2 * CP + C]

    x1c = x1[PAD + IMG:PAD + IMG + T, :]
    g1 = jnp.maximum(_bdot(x1c.astype(BF16), wg1t_ref[...]) + bg1_ref[...], 0.0)
    g2 = jax.nn.sigmoid(
        jnp.sum(wg2_ref[...].astype(F32) * g1.astype(BF16).astype(F32),
                axis=1, keepdims=True) + bg2_ref[...])

    @pl.when(i == 0)
    def _init():
        sqq_ref[...] = jnp.zeros_like(sqq_ref)
        sqk_ref[...] = jnp.zeros_like(sqk_ref)
        gs_ref[...] = jnp.zeros_like(gs_ref)

    sqq_ref[...] += jnp.sum(q * q, axis=0, keepdims=True)
    sqk_ref[...] += jnp.sum(k * k, axis=0, keepdims=True)
    gs_ref[...] += jnp.sum(g2, keepdims=True)


def _gram_kernel(q_ref, k_ref, sqq_ref, sqk_ref, g_ref):
    i = pl.program_id(0)
    nq = jnp.maximum(jnp.sqrt(sqq_ref[...]), 1e-12)
    nk = jnp.maximum(jnp.sqrt(sqk_ref[...]), 1e-12)
    qn = (q_ref[...] / nq).astype(BF16)
    kn = (k_ref[...] / nk).astype(BF16)

    @pl.when(i == 0)
    def _init():
        g_ref[...] = jnp.zeros_like(g_ref)

    g_ref[...] += lax.dot_general(qn, kn, (((0,), (0,)), ((), ())),
                                  preferred_element_type=F32)


def _mask_kernel(g_ref, gs_ref, tv_ref, a_ref):
    attn = g_ref[...] * tv_ref[...]
    blocks = [attn[h * HD:(h + 1) * HD, h * HD:(h + 1) * HD] for h in range(H)]
    b = jnp.concatenate(blocks, axis=0)
    dkf = jnp.clip(jnp.floor(HD * gs_ref[0, 0] / N), 1.0, float(HD))
    bd = b[:, :, None]
    be = b[:, None, :]
    ie = lax.broadcasted_iota(jnp.int32, (C, HD, HD), 2)
    idx = lax.broadcasted_iota(jnp.int32, (C, HD, HD), 1)
    gt = (be > bd).astype(F32)
    eq = ((be == bd) & (ie < idx)).astype(F32)
    rank = jnp.sum(gt + eq, axis=2)
    keep = rank < dkf
    keepf = keep.astype(F32)
    bm = jnp.where(keep, b, NEG)
    m = jnp.max(bm, axis=1, keepdims=True)
    e = jnp.exp(bm - m) * keepf
    s = jnp.sum(e, axis=1, keepdims=True)
    a = e / s
    at = jnp.concatenate([a.T] * H, axis=0)
    ic = lax.broadcasted_iota(jnp.int32, (C, C), 0) // HD
    jc = lax.broadcasted_iota(jnp.int32, (C, C), 1) // HD
    a_ref[...] = jnp.where(ic == jc, at, 0.0)


def _out_kernel(at_ref, v_ref, a1_ref, a2_ref, a3_ref, a4_ref, o_ref):
    o = _bdot(v_ref[...].astype(BF16), at_ref[...].astype(BF16))
    o_ref[...] = (o * a1_ref[0, 0] + o * a2_ref[0, 0]
                  + o * a3_ref[0, 0] + o * a4_ref[0, 0])


def kernel(x, W_lin0, b_lin0, W_qkv, W_dw, W_g1, b_g1, W_g2, b_g2,
           temperature, attn1, attn2, attn3, attn4):
    xt = jnp.transpose(x.reshape(C, N))
    wlt = jnp.transpose(W_lin0).astype(BF16)
    wq3 = W_qkv.reshape(3, C, C)
    wqt = jnp.zeros((C, 3 * CP), F32)
    w9p = jnp.zeros((9, 3 * CP), F32)
    wdw9 = jnp.transpose(W_dw.reshape(3 * C, 9))
    for part in range(3):
        wqt = wqt.at[:, part * CP:part * CP + C].set(jnp.transpose(wq3[part]))
        w9p = w9p.at[:, part * CP:part * CP + C].set(
            wdw9[:, part * C:(part + 1) * C])
    wqt = wqt.astype(BF16)
    wg1t = jnp.transpose(W_g1).astype(BF16)
    wg2 = W_g2.reshape(1, C // 2).astype(BF16)
    blin = b_lin0.reshape(1, C)
    bg1 = b_g1.reshape(1, C // 2)
    bg2 = b_g2.reshape(1, 1)
    tv = jnp.repeat(temperature.reshape(H, 1), HD, axis=0)
    sc = lambda a: a.reshape(1, 1)

    full = lambda s: pl.BlockSpec(s, lambda i: (0, 0))
    tile = pl.BlockSpec((T, C), lambda i: (i, 0))
    q, k, v, sqq, sqk, gs = pl.pallas_call(
        _pass1_kernel,
        grid=(NT,),
        in_specs=[
            tile,
            pl.BlockSpec((IMG, C), lambda i: (jnp.maximum(i * R - 1, 0), 0)),
            pl.BlockSpec((IMG, C), lambda i: (jnp.minimum((i + 1) * R, IMG - 1), 0)),
            full((C, C)),
            full((1, C)),
            full((C, 3 * CP)),
            full((9, 3 * CP)),
            full((C, C // 2)),
            full((1, C // 2)),
            full((1, C // 2)),
            full((1, 1)),
        ],
        out_specs=[tile, tile, tile, full((1, C)), full((1, C)), full((1, 1))],
        out_shape=[
            jax.ShapeDtypeStruct((N, C), F32),
            jax.ShapeDtypeStruct((N, C), F32),
            jax.ShapeDtypeStruct((N, C), F32),
            jax.ShapeDtypeStruct((1, C), F32),
            jax.ShapeDtypeStruct((1, C), F32),
            jax.ShapeDtypeStruct((1, 1), F32),
        ],
    )(xt, xt, xt, wlt, blin, wqt, w9p, wg1t, bg1, wg2, bg2)

    g = pl.pallas_call(
        _gram_kernel,
        grid=(NT,),
        in_specs=[tile, tile, full((1, C)), full((1, C))],
        out_specs=full((C, C)),
        out_shape=jax.ShapeDtypeStruct((C, C), F32),
    )(q, k, sqq, sqk)

    a_t = pl.pallas_call(
        _mask_kernel,
        out_shape=jax.ShapeDtypeStruct((C, C), F32),
    )(g, gs, tv)

    o = pl.pallas_call(
        _out_kernel,
        grid=(NT,),
        in_specs=[full((C, C)), tile, full((1, 1)), full((1, 1)),
                  full((1, 1)), full((1, 1))],
        out_specs=tile,
        out_shape=jax.ShapeDtypeStruct((N, C), F32),
    )(a_t, v, sc(attn1), sc(attn2), sc(attn3), sc(attn4))

    return jnp.transpose(o).reshape(1, C, IMG, IMG)


# speedup vs baseline: 1.5456x; 1.5456x over previous
import jax
import jax.numpy as jnp
from jax import lax
from jax.experimental import pallas as pl

C = 192
H = 8
HD = C // H
IMG = 224
N = IMG * IMG
R = 8
T = R * IMG
NT = IMG // R
PAD = 8
W_EXT = T + 2 * IMG + 2 * PAD
CP = 256
NEG = -1e30

F32 = jnp.float32
BF16 = jnp.bfloat16


def _bdot(a, b):
    return jnp.dot(a, b, preferred_element_type=F32)


def _pass1_kernel(xc_ref, xt_ref, xb_ref, wlt_ref, bl_ref, wqt_ref, w9_ref,
                  wg1t_ref, bg1_ref, wg2_ref, bg2_ref,
                  q_ref, k_ref, v_ref, sqq_ref, sqk_ref, gs_ref):
    i = pl.program_id(0)
    zpad = jnp.zeros((PAD, C), BF16)
    xe = jnp.concatenate([zpad, xt_ref[...].astype(BF16),
                          xc_ref[...].astype(BF16),
                          xb_ref[...].astype(BF16), zpad], axis=0)
    x1 = _bdot(xe, wlt_ref[...]) + bl_ref[...]
    pre = _bdot(x1.astype(BF16), wqt_ref[...])
    rows = lax.broadcasted_iota(jnp.int32, (W_EXT, 1), 0)
    top_ok = (rows >= PAD + IMG) | (i > 0)
    bot_ok = (rows < W_EXT - PAD - IMG) | (i < NT - 1)
    pad_ok = (rows >= PAD) & (rows < W_EXT - PAD)
    pre = pre * (top_ok & bot_ok & pad_ok).astype(F32)

    preb = pre.astype(BF16).astype(F32)
    shift = {dx: preb[PAD + dx:PAD + dx + T + 2 * IMG, :] for dx in (-1, 0, 1)}
    col = lax.broadcasted_iota(jnp.int32, (T, 1), 0) % IMG
    lm = (col != 0).astype(F32)
    rm = (col != IMG - 1).astype(F32)
    w9 = w9_ref[...]

    def tap(dy, dx):
        j = (dy + 1) * 3 + (dx + 1)
        s = IMG + dy * IMG
        t = w9[j:j + 1, :] * shift[dx][s:s + T, :]
        if dx == -1:
            t = t * lm
        elif dx == 1:
            t = t * rm
        return t

    y = tap(-1, -1)
    for dy, dx in [(-1, 0), (-1, 1), (0, -1), (0, 0), (0, 1),
                   (1, -1), (1, 0), (1, 1)]:
        y = y + tap(dy, dx)

    q = y[:, 0:C]
    k = y[:, CP:CP + C]
    q_ref[...] = q
    k_ref[...] = k
    v_ref[...] = y[:, 2 * CP:2 * CP + C]

    x1c = x1[PAD + IMG:PAD + IMG + T, :]
    g1 = jnp.maximum(_bdot(x1c.astype(BF16), wg1t_ref[...]) + bg1_ref[...], 0.0)
    g2 = jax.nn.sigmoid(
        jnp.sum(wg2_ref[...].astype(F32) * g1.astype(BF16).astype(F32),
                axis=1, keepdims=True) + bg2_ref[...])

    @pl.when(i == 0)
    def _init():
        sqq_ref[...] = jnp.zeros_like(sqq_ref)
        sqk_ref[...] = jnp.zeros_like(sqk_ref)
        gs_ref[...] = jnp.zeros_like(gs_ref)

    sqq_ref[...] += jnp.sum(q * q, axis=0, keepdims=True)
    sqk_ref[...] += jnp.sum(k * k, axis=0, keepdims=True)
    gs_ref[...] += jnp.sum(g2, keepdims=True)


def _gram_kernel(q_ref, k_ref, sqq_ref, sqk_ref, g_ref):
    i = pl.program_id(0)
    nq = jnp.maximum(jnp.sqrt(sqq_ref[...]), 1e-12)
    nk = jnp.maximum(jnp.sqrt(sqk_ref[...]), 1e-12)
    qn = (q_ref[...] / nq).astype(BF16)
    kn = (k_ref[...] / nk).astype(BF16)

    @pl.when(i == 0)
    def _init():
        g_ref[...] = jnp.zeros_like(g_ref)

    g_ref[...] += lax.dot_general(qn, kn, (((0,), (0,)), ((), ())),
                                  preferred_element_type=F32)


def _mask_kernel(g_ref, gs_ref, tv_ref, a_ref):
    attn = g_ref[...] * tv_ref[...]
    blocks = [attn[h * HD:(h + 1) * HD, h * HD:(h + 1) * HD] for h in range(H)]
    b = jnp.concatenate(blocks, axis=0)
    dkf = jnp.clip(jnp.floor(HD * gs_ref[0, 0] / N), 1.0, float(HD))
    bd = b[:, :, None]
    be = b[:, None, :]
    ie = lax.broadcasted_iota(jnp.int32, (C, HD, HD), 2)
    idx = lax.broadcasted_iota(jnp.int32, (C, HD, HD), 1)
    gt = (be > bd).astype(F32)
    eq = ((be == bd) & (ie < idx)).astype(F32)
    rank = jnp.sum(gt + eq, axis=2)
    keep = rank < dkf
    keepf = keep.astype(F32)
    bm = jnp.where(keep, b, NEG)
    m = jnp.max(bm, axis=1, keepdims=True)
    e = jnp.exp(bm - m) * keepf
    s = jnp.sum(e, axis=1, keepdims=True)
    a = e / s
    at = jnp.concatenate([a.T] * H, axis=0)
    ic = lax.broadcasted_iota(jnp.int32, (C, C), 0) // HD
    jc = lax.broadcasted_iota(jnp.int32, (C, C), 1) // HD
    a_ref[...] = jnp.where(ic == jc, at, 0.0)


def _out_kernel(at_ref, v_ref, a1_ref, a2_ref, a3_ref, a4_ref, o_ref):
    o = _bdot(v_ref[...].astype(BF16), at_ref[...].astype(BF16))
    o_ref[...] = (o * a1_ref[0, 0] + o * a2_ref[0, 0]
                  + o * a3_ref[0, 0] + o * a4_ref[0, 0])


def kernel(x, W_lin0, b_lin0, W_qkv, W_dw, W_g1, b_g1, W_g2, b_g2,
           temperature, attn1, attn2, attn3, attn4):
    xt = x.reshape(N, C)
    wlt = jnp.transpose(W_lin0).astype(BF16)
    wq3 = W_qkv.reshape(3, C, C)
    wqt = jnp.zeros((C, 3 * CP), F32)
    w9p = jnp.zeros((9, 3 * CP), F32)
    wdw9 = jnp.transpose(W_dw.reshape(3 * C, 9))
    for part in range(3):
        wqt = wqt.at[:, part * CP:part * CP + C].set(jnp.transpose(wq3[part]))
        w9p = w9p.at[:, part * CP:part * CP + C].set(
            wdw9[:, part * C:(part + 1) * C])
    wqt = wqt.astype(BF16)
    wg1t = jnp.transpose(W_g1).astype(BF16)
    wg2 = W_g2.reshape(1, C // 2).astype(BF16)
    blin = b_lin0.reshape(1, C)
    bg1 = b_g1.reshape(1, C // 2)
    bg2 = b_g2.reshape(1, 1)
    tv = jnp.repeat(temperature.reshape(H, 1), HD, axis=0)
    sc = lambda a: a.reshape(1, 1)

    full = lambda s: pl.BlockSpec(s, lambda i: (0, 0))
    tile = pl.BlockSpec((T, C), lambda i: (i, 0))
    q, k, v, sqq, sqk, gs = pl.pallas_call(
        _pass1_kernel,
        grid=(NT,),
        in_specs=[
            tile,
            pl.BlockSpec((IMG, C), lambda i: (jnp.maximum(i * R - 1, 0), 0)),
            pl.BlockSpec((IMG, C), lambda i: (jnp.minimum((i + 1) * R, IMG - 1), 0)),
            full((C, C)),
            full((1, C)),
            full((C, 3 * CP)),
            full((9, 3 * CP)),
            full((C, C // 2)),
            full((1, C // 2)),
            full((1, C // 2)),
            full((1, 1)),
        ],
        out_specs=[tile, tile, tile, full((1, C)), full((1, C)), full((1, 1))],
        out_shape=[
            jax.ShapeDtypeStruct((N, C), F32),
            jax.ShapeDtypeStruct((N, C), F32),
            jax.ShapeDtypeStruct((N, C), F32),
            jax.ShapeDtypeStruct((1, C), F32),
            jax.ShapeDtypeStruct((1, C), F32),
            jax.ShapeDtypeStruct((1, 1), F32),
        ],
    )(xt, xt, xt, wlt, blin, wqt, w9p, wg1t, bg1, wg2, bg2)

    g = pl.pallas_call(
        _gram_kernel,
        grid=(NT,),
        in_specs=[tile, tile, full((1, C)), full((1, C))],
        out_specs=full((C, C)),
        out_shape=jax.ShapeDtypeStruct((C, C), F32),
    )(q, k, sqq, sqk)

    a_t = pl.pallas_call(
        _mask_kernel,
        out_shape=jax.ShapeDtypeStruct((C, C), F32),
    )(g, gs, tv)

    o = pl.pallas_call(
        _out_kernel,
        grid=(NT,),
        in_specs=[full((C, C)), tile, full((1, 1)), full((1, 1)),
                  full((1, 1)), full((1, 1))],
        out_specs=tile,
        out_shape=jax.ShapeDtypeStruct((N, C), F32),
    )(a_t, v, sc(attn1), sc(attn2), sc(attn3), sc(attn4))

    return o.reshape(1, C, IMG, IMG)


# speedup vs baseline: 1.5693x; 1.0153x over previous
import jax
import jax.numpy as jnp
from jax import lax
from jax.experimental import pallas as pl

C = 192
H = 8
HD = C // H
IMG = 224
N = IMG * IMG
R = 8
T = R * IMG
NT = IMG // R
PAD = 8
W_EXT = T + 2 * IMG + 2 * PAD
CP = 256
NEG = -1e30

F32 = jnp.float32
BF16 = jnp.bfloat16


def _bdot(a, b):
    return jnp.dot(a, b, preferred_element_type=F32)


def _xpose_kernel(x_ref, o_ref):
    o_ref[...] = jnp.transpose(x_ref[...].astype(BF16))


def _pass1_kernel(xc_ref, xt_ref, xb_ref, wlt_ref, bl_ref, wqt_ref, w9_ref,
                  wg1t_ref, bg1_ref, wg2_ref, bg2_ref,
                  q_ref, k_ref, v_ref, sqq_ref, sqk_ref, gs_ref):
    i = pl.program_id(0)
    zpad = jnp.zeros((PAD, C), BF16)
    xe = jnp.concatenate([zpad, xt_ref[...], xc_ref[...], xb_ref[...],
                          zpad], axis=0)
    x1 = _bdot(xe, wlt_ref[...]) + bl_ref[...]
    pre = _bdot(x1.astype(BF16), wqt_ref[...])
    rows = lax.broadcasted_iota(jnp.int32, (W_EXT, 1), 0)
    top_ok = (rows >= PAD + IMG) | (i > 0)
    bot_ok = (rows < W_EXT - PAD - IMG) | (i < NT - 1)
    pad_ok = (rows >= PAD) & (rows < W_EXT - PAD)
    pre = pre * (top_ok & bot_ok & pad_ok).astype(F32)

    preb = pre.astype(BF16).astype(F32)
    shift = {dx: preb[PAD + dx:PAD + dx + T + 2 * IMG, :] for dx in (-1, 0, 1)}
    col = lax.broadcasted_iota(jnp.int32, (T, 1), 0) % IMG
    lm = (col != 0).astype(F32)
    rm = (col != IMG - 1).astype(F32)
    w9 = w9_ref[...]

    def tap(dy, dx):
        j = (dy + 1) * 3 + (dx + 1)
        s = IMG + dy * IMG
        t = w9[j:j + 1, :] * shift[dx][s:s + T, :]
        if dx == -1:
            t = t * lm
        elif dx == 1:
            t = t * rm
        return t

    y = tap(-1, -1)
    for dy, dx in [(-1, 0), (-1, 1), (0, -1), (0, 0), (0, 1),
                   (1, -1), (1, 0), (1, 1)]:
        y = y + tap(dy, dx)

    q = y[:, 0:C]
    k = y[:, CP:CP + C]
    q_ref[...] = q
    k_ref[...] = k
    v_ref[...] = y[:, 2 * CP:2 * CP + C]

    x1c = x1[PAD + IMG:PAD + IMG + T, :]
    g1 = jnp.maximum(_bdot(x1c.astype(BF16), wg1t_ref[...]) + bg1_ref[...], 0.0)
    g2 = jax.nn.sigmoid(
        jnp.sum(wg2_ref[...].astype(F32) * g1.astype(BF16).astype(F32),
                axis=1, keepdims=True) + bg2_ref[...])

    @pl.when(i == 0)
    def _init():
        sqq_ref[...] = jnp.zeros_like(sqq_ref)
        sqk_ref[...] = jnp.zeros_like(sqk_ref)
        gs_ref[...] = jnp.zeros_like(gs_ref)

    sqq_ref[...] += jnp.sum(q * q, axis=0, keepdims=True)
    sqk_ref[...] += jnp.sum(k * k, axis=0, keepdims=True)
    gs_ref[...] += jnp.sum(g2, keepdims=True)


def _gram_kernel(q_ref, k_ref, sqq_ref, sqk_ref, g_ref):
    i = pl.program_id(0)
    nq = jnp.maximum(jnp.sqrt(sqq_ref[...]), 1e-12)
    nk = jnp.maximum(jnp.sqrt(sqk_ref[...]), 1e-12)
    qn = (q_ref[...] / nq).astype(BF16)
    kn = (k_ref[...] / nk).astype(BF16)

    @pl.when(i == 0)
    def _init():
        g_ref[...] = jnp.zeros_like(g_ref)

    g_ref[...] += lax.dot_general(qn, kn, (((0,), (0,)), ((), ())),
                                  preferred_element_type=F32)


def _mask_kernel(g_ref, gs_ref, tv_ref, a_ref):
    attn = g_ref[...] * tv_ref[...]
    blocks = [attn[h * HD:(h + 1) * HD, h * HD:(h + 1) * HD] for h in range(H)]
    b = jnp.concatenate(blocks, axis=0)
    dkf = jnp.clip(jnp.floor(HD * gs_ref[0, 0] / N), 1.0, float(HD))
    bd = b[:, :, None]
    be = b[:, None, :]
    ie = lax.broadcasted_iota(jnp.int32, (C, HD, HD), 2)
    idx = lax.broadcasted_iota(jnp.int32, (C, HD, HD), 1)
    gt = (be > bd).astype(F32)
    eq = ((be == bd) & (ie < idx)).astype(F32)
    rank = jnp.sum(gt + eq, axis=2)
    keep = rank < dkf
    keepf = keep.astype(F32)
    bm = jnp.where(keep, b, NEG)
    m = jnp.max(bm, axis=1, keepdims=True)
    e = jnp.exp(bm - m) * keepf
    s = jnp.sum(e, axis=1, keepdims=True)
    a = e / s
    at = jnp.concatenate([a] * H, axis=1)
    ic = lax.broadcasted_iota(jnp.int32, (C, C), 0) // HD
    jc = lax.broadcasted_iota(jnp.int32, (C, C), 1) // HD
    a_ref[...] = jnp.where(ic == jc, at, 0.0)


def _out_kernel(a_ref, v_ref, a1_ref, a2_ref, a3_ref, a4_ref, o_ref):
    o = lax.dot_general(a_ref[...].astype(BF16), v_ref[...].astype(BF16),
                        (((1,), (1,)), ((), ())), preferred_element_type=F32)
    o_ref[...] = (o * a1_ref[0, 0] + o * a2_ref[0, 0]
                  + o * a3_ref[0, 0] + o * a4_ref[0, 0])


def kernel(x, W_lin0, b_lin0, W_qkv, W_dw, W_g1, b_g1, W_g2, b_g2,
           temperature, attn1, attn2, attn3, attn4):
    x2 = x.reshape(C, N)
    xt = pl.pallas_call(
        _xpose_kernel,
        grid=(NT,),
        in_specs=[pl.BlockSpec((C, T), lambda i: (0, i))],
        out_specs=pl.BlockSpec((T, C), lambda i: (i, 0)),
        out_shape=jax.ShapeDtypeStruct((N, C), BF16),
    )(x2)
    wlt = jnp.transpose(W_lin0).astype(BF16)
    wq3 = W_qkv.reshape(3, C, C)
    wqt = jnp.zeros((C, 3 * CP), F32)
    w9p = jnp.zeros((9, 3 * CP), F32)
    wdw9 = jnp.transpose(W_dw.reshape(3 * C, 9))
    for part in range(3):
        wqt = wqt.at[:, part * CP:part * CP + C].set(jnp.transpose(wq3[part]))
        w9p = w9p.at[:, part * CP:part * CP + C].set(
            wdw9[:, part * C:(part + 1) * C])
    wqt = wqt.astype(BF16)
    wg1t = jnp.transpose(W_g1).astype(BF16)
    wg2 = W_g2.reshape(1, C // 2).astype(BF16)
    blin = b_lin0.reshape(1, C)
    bg1 = b_g1.reshape(1, C // 2)
    bg2 = b_g2.reshape(1, 1)
    tv = jnp.repeat(temperature.reshape(H, 1), HD, axis=0)
    sc = lambda a: a.reshape(1, 1)

    full = lambda s: pl.BlockSpec(s, lambda i: (0, 0))
    tile = pl.BlockSpec((T, C), lambda i: (i, 0))
    q, k, v, sqq, sqk, gs = pl.pallas_call(
        _pass1_kernel,
        grid=(NT,),
        in_specs=[
            tile,
            pl.BlockSpec((IMG, C), lambda i: (jnp.maximum(i * R - 1, 0), 0)),
            pl.BlockSpec((IMG, C), lambda i: (jnp.minimum((i + 1) * R, IMG - 1), 0)),
            full((C, C)),
            full((1, C)),
            full((C, 3 * CP)),
            full((9, 3 * CP)),
            full((C, C // 2)),
            full((1, C // 2)),
            full((1, C // 2)),
            full((1, 1)),
        ],
        out_specs=[tile, tile, tile, full((1, C)), full((1, C)), full((1, 1))],
        out_shape=[
            jax.ShapeDtypeStruct((N, C), F32),
            jax.ShapeDtypeStruct((N, C), F32),
            jax.ShapeDtypeStruct((N, C), F32),
            jax.ShapeDtypeStruct((1, C), F32),
            jax.ShapeDtypeStruct((1, C), F32),
            jax.ShapeDtypeStruct((1, 1), F32),
        ],
    )(xt, xt, xt, wlt, blin, wqt, w9p, wg1t, bg1, wg2, bg2)

    g = pl.pallas_call(
        _gram_kernel,
        grid=(NT,),
        in_specs=[tile, tile, full((1, C)), full((1, C))],
        out_specs=full((C, C)),
        out_shape=jax.ShapeDtypeStruct((C, C), F32),
    )(q, k, sqq, sqk)

    a_t = pl.pallas_call(
        _mask_kernel,
        out_shape=jax.ShapeDtypeStruct((C, C), F32),
    )(g, gs, tv)

    o = pl.pallas_call(
        _out_kernel,
        grid=(NT,),
        in_specs=[full((C, C)), tile, full((1, 1)), full((1, 1)),
                  full((1, 1)), full((1, 1))],
        out_specs=pl.BlockSpec((C, T), lambda i: (0, i)),
        out_shape=jax.ShapeDtypeStruct((C, N), F32),
    )(a_t, v, sc(attn1), sc(attn2), sc(attn3), sc(attn4))

    return o.reshape(1, C, IMG, IMG)


# speedup vs baseline: 1.5731x; 1.0025x over previous
import jax
import jax.numpy as jnp
from jax import lax
from jax.experimental import pallas as pl

C = 192
H = 8
HD = C // H
IMG = 224
N = IMG * IMG
R = 8
T = R * IMG
NT = IMG // R
PAD = 8
W_EXT = T + 2 * IMG + 2 * PAD
CP = 256
NEG = -1e30

F32 = jnp.float32
BF16 = jnp.bfloat16


def _bdot(a, b):
    return jnp.dot(a, b, preferred_element_type=F32)


def _xpose_kernel(x_ref, o_ref):
    o_ref[...] = jnp.transpose(x_ref[...].astype(BF16))


def _pass1_kernel(xc_ref, xt_ref, xb_ref, wlt_ref, bl_ref, wqt_ref, w9_ref,
                  wg1t_ref, bg1_ref, wg2_ref, bg2_ref,
                  q_ref, k_ref, v_ref, sqq_ref, sqk_ref, gs_ref):
    i = pl.program_id(0)
    zpad = jnp.zeros((PAD, C), BF16)
    xe = jnp.concatenate([zpad, xt_ref[...], xc_ref[...], xb_ref[...],
                          zpad], axis=0)
    x1 = _bdot(xe, wlt_ref[...]) + bl_ref[...]
    pre = _bdot(x1.astype(BF16), wqt_ref[...])
    rows = lax.broadcasted_iota(jnp.int32, (W_EXT, 1), 0)
    top_ok = (rows >= PAD + IMG) | (i > 0)
    bot_ok = (rows < W_EXT - PAD - IMG) | (i < NT - 1)
    pad_ok = (rows >= PAD) & (rows < W_EXT - PAD)
    pre = pre * (top_ok & bot_ok & pad_ok).astype(F32)

    preb = pre.astype(BF16).astype(F32)
    shift = {dx: preb[PAD + dx:PAD + dx + T + 2 * IMG, :] for dx in (-1, 0, 1)}
    col = lax.broadcasted_iota(jnp.int32, (T, 1), 0) % IMG
    lm = (col != 0).astype(F32)
    rm = (col != IMG - 1).astype(F32)
    w9 = w9_ref[...]

    def tap(part, dy, dx):
        j = (dy + 1) * 3 + (dx + 1)
        s = IMG + dy * IMG
        t = (w9[j:j + 1, part * CP:part * CP + C]
             * shift[dx][s:s + T, part * CP:part * CP + C])
        if dx == -1:
            t = t * lm
        elif dx == 1:
            t = t * rm
        return t

    taps = [(-1, -1), (-1, 0), (-1, 1), (0, -1), (0, 0), (0, 1),
            (1, -1), (1, 0), (1, 1)]

    def dwconv(part):
        y = tap(part, *taps[0])
        for dy, dx in taps[1:]:
            y = y + tap(part, dy, dx)
        return y

    q = dwconv(0)
    k = dwconv(1)
    q_ref[...] = q
    k_ref[...] = k
    v_ref[...] = dwconv(2).astype(BF16)

    x1c = x1[PAD + IMG:PAD + IMG + T, :]
    g1 = jnp.maximum(_bdot(x1c.astype(BF16), wg1t_ref[...]) + bg1_ref[...], 0.0)
    g2 = jax.nn.sigmoid(
        jnp.sum(wg2_ref[...].astype(F32) * g1.astype(BF16).astype(F32),
                axis=1, keepdims=True) + bg2_ref[...])

    @pl.when(i == 0)
    def _init():
        sqq_ref[...] = jnp.zeros_like(sqq_ref)
        sqk_ref[...] = jnp.zeros_like(sqk_ref)
        gs_ref[...] = jnp.zeros_like(gs_ref)

    sqq_ref[...] += jnp.sum(q * q, axis=0, keepdims=True)
    sqk_ref[...] += jnp.sum(k * k, axis=0, keepdims=True)
    gs_ref[...] += jnp.sum(g2, keepdims=True)


def _gram_kernel(q_ref, k_ref, sqq_ref, sqk_ref, g_ref):
    i = pl.program_id(0)
    nq = jnp.maximum(jnp.sqrt(sqq_ref[...]), 1e-12)
    nk = jnp.maximum(jnp.sqrt(sqk_ref[...]), 1e-12)
    qn = (q_ref[...] / nq).astype(BF16)
    kn = (k_ref[...] / nk).astype(BF16)

    @pl.when(i == 0)
    def _init():
        g_ref[...] = jnp.zeros_like(g_ref)

    g_ref[...] += lax.dot_general(qn, kn, (((0,), (0,)), ((), ())),
                                  preferred_element_type=F32)


def _mask_kernel(g_ref, gs_ref, tv_ref, a_ref):
    attn = g_ref[...] * tv_ref[...]
    blocks = [attn[h * HD:(h + 1) * HD, h * HD:(h + 1) * HD] for h in range(H)]
    b = jnp.concatenate(blocks, axis=0)
    dkf = jnp.clip(jnp.floor(HD * gs_ref[0, 0] / N), 1.0, float(HD))
    bd = b[:, :, None]
    be = b[:, None, :]
    ie = lax.broadcasted_iota(jnp.int32, (C, HD, HD), 2)
    idx = lax.broadcasted_iota(jnp.int32, (C, HD, HD), 1)
    gt = (be > bd).astype(F32)
    eq = ((be == bd) & (ie < idx)).astype(F32)
    rank = jnp.sum(gt + eq, axis=2)
    keep = rank < dkf
    keepf = keep.astype(F32)
    bm = jnp.where(keep, b, NEG)
    m = jnp.max(bm, axis=1, keepdims=True)
    e = jnp.exp(bm - m) * keepf
    s = jnp.sum(e, axis=1, keepdims=True)
    a = e / s
    at = jnp.concatenate([a] * H, axis=1)
    ic = lax.broadcasted_iota(jnp.int32, (C, C), 0) // HD
    jc = lax.broadcasted_iota(jnp.int32, (C, C), 1) // HD
    a_ref[...] = jnp.where(ic == jc, at, 0.0)


def _out_kernel(a_ref, v_ref, a1_ref, a2_ref, a3_ref, a4_ref, o_ref):
    o = lax.dot_general(a_ref[...].astype(BF16), v_ref[...],
                        (((1,), (1,)), ((), ())), preferred_element_type=F32)
    o_ref[...] = (o * a1_ref[0, 0] + o * a2_ref[0, 0]
                  + o * a3_ref[0, 0] + o * a4_ref[0, 0])


def kernel(x, W_lin0, b_lin0, W_qkv, W_dw, W_g1, b_g1, W_g2, b_g2,
           temperature, attn1, attn2, attn3, attn4):
    x2 = x.reshape(C, N)
    xt = pl.pallas_call(
        _xpose_kernel,
        grid=(NT,),
        in_specs=[pl.BlockSpec((C, T), lambda i: (0, i))],
        out_specs=pl.BlockSpec((T, C), lambda i: (i, 0)),
        out_shape=jax.ShapeDtypeStruct((N, C), BF16),
    )(x2)
    wlt = jnp.transpose(W_lin0).astype(BF16)
    wq3 = W_qkv.reshape(3, C, C)
    wqt = jnp.zeros((C, 3 * CP), F32)
    w9p = jnp.zeros((9, 3 * CP), F32)
    wdw9 = jnp.transpose(W_dw.reshape(3 * C, 9))
    for part in range(3):
        wqt = wqt.at[:, part * CP:part * CP + C].set(jnp.transpose(wq3[part]))
        w9p = w9p.at[:, part * CP:part * CP + C].set(
            wdw9[:, part * C:(part + 1) * C])
    wqt = wqt.astype(BF16)
    wg1t = jnp.transpose(W_g1).astype(BF16)
    wg2 = W_g2.reshape(1, C // 2).astype(BF16)
    blin = b_lin0.reshape(1, C)
    bg1 = b_g1.reshape(1, C // 2)
    bg2 = b_g2.reshape(1, 1)
    tv = jnp.repeat(temperature.reshape(H, 1), HD, axis=0)
    sc = lambda a: a.reshape(1, 1)

    full = lambda s: pl.BlockSpec(s, lambda i: (0, 0))
    tile = pl.BlockSpec((T, C), lambda i: (i, 0))
    q, k, v, sqq, sqk, gs = pl.pallas_call(
        _pass1_kernel,
        grid=(NT,),
        in_specs=[
            tile,
            pl.BlockSpec((IMG, C), lambda i: (jnp.maximum(i * R - 1, 0), 0)),
            pl.BlockSpec((IMG, C), lambda i: (jnp.minimum((i + 1) * R, IMG - 1), 0)),
            full((C, C)),
            full((1, C)),
            full((C, 3 * CP)),
            full((9, 3 * CP)),
            full((C, C // 2)),
            full((1, C // 2)),
            full((1, C // 2)),
            full((1, 1)),
        ],
        out_specs=[tile, tile, tile, full((1, C)), full((1, C)), full((1, 1))],
        out_shape=[
            jax.ShapeDtypeStruct((N, C), F32),
            jax.ShapeDtypeStruct((N, C), F32),
            jax.ShapeDtypeStruct((N, C), BF16),
            jax.ShapeDtypeStruct((1, C), F32),
            jax.ShapeDtypeStruct((1, C), F32),
            jax.ShapeDtypeStruct((1, 1), F32),
        ],
    )(xt, xt, xt, wlt, blin, wqt, w9p, wg1t, bg1, wg2, bg2)

    g = pl.pallas_call(
        _gram_kernel,
        grid=(NT,),
        in_specs=[tile, tile, full((1, C)), full((1, C))],
        out_specs=full((C, C)),
        out_shape=jax.ShapeDtypeStruct((C, C), F32),
    )(q, k, sqq, sqk)

    a_t = pl.pallas_call(
        _mask_kernel,
        out_shape=jax.ShapeDtypeStruct((C, C), F32),
    )(g, gs, tv)

    o = pl.pallas_call(
        _out_kernel,
        grid=(NT,),
        in_specs=[full((C, C)), tile, full((1, 1)), full((1, 1)),
                  full((1, 1)), full((1, 1))],
        out_specs=pl.BlockSpec((C, T), lambda i: (0, i)),
        out_shape=jax.ShapeDtypeStruct((C, N), F32),
    )(a_t, v, sc(attn1), sc(attn2), sc(attn3), sc(attn4))

    return o.reshape(1, C, IMG, IMG)
